# baseline (speedup 1.0000x reference)
import functools

import jax
import jax.numpy as jnp
from jax import lax
from jax.experimental import pallas as pl
from jax.experimental.pallas import tpu as pltpu
from jax.experimental.pallas import tpu_sc as plsc

N = 10000
E = 320000
D = 128
NPAD = 10240
CHUNK = 128
TILES = 32
CPT = 80
EPT = CPT * CHUNK
IDX_ROWS = TILES * CPT
DUMP = N
RPT = NPAD // 16
KB = 8
NBLK = CPT // KB
BLK = 512
GRID = -(-NPAD // BLK)


def _vmesh():
    return plsc.VectorSubcoreMesh(core_axis_name="c", subcore_axis_name="s")


def _sc_degree(col2d):
    @functools.partial(
        pl.kernel,
        out_type=jax.ShapeDtypeStruct((2 * NPAD, 16), jnp.float32),
        mesh=_vmesh(),
        scratch_types=[
            pltpu.VMEM_SHARED((NPAD, 16), jnp.float32),
            pltpu.VMEM((128, 16), jnp.float32),
            pltpu.VMEM((CHUNK, 16), jnp.float32),
            pltpu.VMEM((CPT, CHUNK), jnp.int32),
            pltpu.SemaphoreType.DMA,
        ],
    )
    def deg_kernel(col_hbm, out_hbm, dacc, zbuf, ones, coli, sem):
        c = lax.axis_index("c")
        s = lax.axis_index("s")
        wid = s * 2 + c

        @pl.loop(0, 128)
        def _(i):
            zbuf[i] = jnp.zeros((16,), jnp.float32)

        @pl.loop(0, CHUNK)
        def _(i):
            ones[i] = jnp.ones((16,), jnp.float32)

        @pl.loop(0, RPT // 128)
        def _(k):
            pltpu.sync_copy(zbuf, dacc.at[pl.ds(s * RPT + k * 128, 128)])

        plsc.subcore_barrier()
        pltpu.sync_copy(col_hbm.at[pl.ds(wid * CPT, CPT)], coli)

        @pl.loop(0, CPT)
        def _(t):
            pltpu.async_copy(ones, dacc.at[coli.at[t]], sem, add=True)

        @pl.loop(0, CPT)
        def _(t):
            pltpu.make_async_copy(ones, dacc.at[coli.at[0]], sem).wait()

        plsc.subcore_barrier()
        pltpu.sync_copy(
            dacc.at[pl.ds(s * RPT, RPT)],
            out_hbm.at[pl.ds(c * NPAD + s * RPT, RPT)],
        )

    return deg_kernel(col2d)


def _sc_scatter(row1d, col1d, h2):
    @functools.partial(
        pl.kernel,
        out_type=jax.ShapeDtypeStruct((2 * NPAD, D), jnp.float32),
        mesh=_vmesh(),
        scratch_types=[
            pltpu.VMEM_SHARED((NPAD, D), jnp.float32),
            pltpu.VMEM((CHUNK,), jnp.int32),
            pltpu.VMEM((CHUNK,), jnp.int32),
            pltpu.VMEM((CHUNK, D), jnp.float32),
            pltpu.SemaphoreType.DMA,
        ],
    )
    def scat_kernel(row_hbm, col_hbm, h2_hbm, out_hbm, acc, rowi, coli, rows, sem):
        c = lax.axis_index("c")
        s = lax.axis_index("s")
        wid = s * 2 + c

        @pl.loop(0, CHUNK)
        def _(i):
            @pl.loop(0, D, step=16)
            def _(j):
                rows[i, pl.ds(j, 16)] = jnp.zeros((16,), jnp.float32)

        @pl.loop(0, RPT // CHUNK)
        def _(k):
            pltpu.sync_copy(rows, acc.at[pl.ds(s * RPT + k * CHUNK, CHUNK)])

        plsc.subcore_barrier()

        @pl.loop(0, CPT)
        def _(t):
            base = wid * EPT + t * CHUNK
            pltpu.sync_copy(row_hbm.at[pl.ds(base, CHUNK)], rowi)
            pltpu.sync_copy(col_hbm.at[pl.ds(base, CHUNK)], coli)
            pltpu.async_copy(h2_hbm.at[rowi], rows, sem).wait()
            pltpu.sync_copy(rows, acc.at[coli], add=True)

        plsc.subcore_barrier()
        pltpu.sync_copy(
            acc.at[pl.ds(s * RPT, RPT)],
            out_hbm.at[pl.ds(c * NPAD + s * RPT, RPT)],
        )

    return scat_kernel(row1d, col1d, h2)


def _tc_matmul(x, w):
    def body(x_ref, w_ref, h_ref):
        h_ref[...] = jnp.dot(x_ref[...], w_ref[...],
                             preferred_element_type=jnp.float32)

    return pl.pallas_call(
        body,
        grid=(GRID,),
        in_specs=[
            pl.BlockSpec((BLK, 128), lambda i: (i, 0)),
            pl.BlockSpec((128, 128), lambda i: (0, 0)),
        ],
        out_specs=pl.BlockSpec((BLK, 128), lambda i: (i, 0)),
        out_shape=jax.ShapeDtypeStruct((N, 128), jnp.float32),
    )(x, w)


def _tc_scale(h, degp):
    def body(h_ref, d0_ref, d1_ref, h2_ref):
        d = d0_ref[:, 0:1] + d1_ref[:, 0:1] + 1.0
        h2_ref[...] = h_ref[...] * lax.rsqrt(d)

    return pl.pallas_call(
        body,
        grid=(GRID,),
        in_specs=[
            pl.BlockSpec((BLK, 128), lambda i: (i, 0)),
            pl.BlockSpec((BLK, 16), lambda i: (i, 0)),
            pl.BlockSpec((BLK, 16), lambda i: (i + GRID, 0)),
        ],
        out_specs=pl.BlockSpec((BLK, 128), lambda i: (i, 0)),
        out_shape=jax.ShapeDtypeStruct((N, 128), jnp.float32),
    )(h, degp, degp)


def _tc_final(tmp, h2, degp, bmu, bls):
    def body(t0, t1, h2r, d0, d1, bm, bl, mu_ref, ls_ref):
        d = d0[:, 0:1] + d1[:, 0:1] + 1.0
        dis = lax.rsqrt(d)
        out = dis * (t0[...] + t1[...] + h2r[...])
        mu_ref[...] = out[:, :64] + bm[0:1, :]
        ls_ref[...] = out[:, 64:] + bl[0:1, :]

    return pl.pallas_call(
        body,
        grid=(GRID,),
        in_specs=[
            pl.BlockSpec((BLK, 128), lambda i: (i, 0)),
            pl.BlockSpec((BLK, 128), lambda i: (i + GRID, 0)),
            pl.BlockSpec((BLK, 128), lambda i: (i, 0)),
            pl.BlockSpec((BLK, 16), lambda i: (i, 0)),
            pl.BlockSpec((BLK, 16), lambda i: (i + GRID, 0)),
            pl.BlockSpec((8, 64), lambda i: (0, 0)),
            pl.BlockSpec((8, 64), lambda i: (0, 0)),
        ],
        out_specs=[
            pl.BlockSpec((BLK, 64), lambda i: (i, 0)),
            pl.BlockSpec((BLK, 64), lambda i: (i, 0)),
        ],
        out_shape=[
            jax.ShapeDtypeStruct((N, 64), jnp.float32),
            jax.ShapeDtypeStruct((N, 64), jnp.float32),
        ],
    )(tmp, tmp, h2, degp, degp, bmu, bls)


def kernel(x, edge_index, W_mu, b_mu, W_logstd, b_logstd):
    w = jnp.concatenate([W_mu, W_logstd], axis=1)
    per_tile = E // TILES
    row_t = edge_index[0].reshape(TILES, per_tile)
    col_t = edge_index[1].reshape(TILES, per_tile)
    padn = EPT - per_tile
    row2d = jnp.pad(row_t, ((0, 0), (0, padn))).reshape(IDX_ROWS, CHUNK)
    col2d = jnp.pad(col_t, ((0, 0), (0, padn)),
                    constant_values=DUMP).reshape(IDX_ROWS, CHUNK)
    bmu = jnp.tile(b_mu[None, :], (8, 1))
    bls = jnp.tile(b_logstd[None, :], (8, 1))
    degp = _sc_degree(col2d)
    h = _tc_matmul(x, w)
    h2 = _tc_scale(h, degp)
    tmp = _sc_scatter(row2d.reshape(-1), col2d.reshape(-1), h2)
    mu, logstd = _tc_final(tmp, h2, degp, bmu, bls)
    return (mu, logstd)


# speedup vs baseline: 1.6348x; 1.6348x over previous
import functools

import jax
import jax.numpy as jnp
from jax import lax
from jax.experimental import pallas as pl
from jax.experimental.pallas import tpu as pltpu
from jax.experimental.pallas import tpu_sc as plsc

N = 10000
E = 320000
D = 128
NPAD = 10240
CHUNK = 128
NUM_CHUNKS = E // CHUNK
TILES = 32
CHUNK_ITERS = -(-NUM_CHUNKS // TILES)
RPT = NPAD // 16
BLK = 512
GRID = -(-NPAD // BLK)


def _vmesh():
    return plsc.VectorSubcoreMesh(core_axis_name="c", subcore_axis_name="s")


def _sc_degree(col):
    @functools.partial(
        pl.kernel,
        out_type=jax.ShapeDtypeStruct((2 * NPAD, 16), jnp.float32),
        mesh=_vmesh(),
        scratch_types=[
            pltpu.VMEM_SHARED((NPAD, 16), jnp.float32),
            pltpu.VMEM((128, 16), jnp.float32),
            pltpu.VMEM((CHUNK, 16), jnp.float32),
            pltpu.VMEM((CHUNK,), jnp.int32),
        ],
    )
    def deg_kernel(col_hbm, out_hbm, dacc, zbuf, ones, coli):
        c = lax.axis_index("c")
        s = lax.axis_index("s")
        wid = s * 2 + c

        @pl.loop(0, 128)
        def _(i):
            zbuf[i] = jnp.zeros((16,), jnp.float32)
            ones[i] = jnp.ones((16,), jnp.float32)

        @pl.loop(0, RPT // 128)
        def _(k):
            pltpu.sync_copy(zbuf, dacc.at[pl.ds(s * RPT + k * 128, 128)])

        plsc.subcore_barrier()

        @pl.loop(0, CHUNK_ITERS)
        def _(i):
            j = wid + i * TILES

            @pl.when(j < NUM_CHUNKS)
            def _():
                pltpu.sync_copy(col_hbm.at[pl.ds(j * CHUNK, CHUNK)], coli)
                pltpu.sync_copy(ones, dacc.at[coli], add=True)

        plsc.subcore_barrier()
        pltpu.sync_copy(
            dacc.at[pl.ds(s * RPT, RPT)],
            out_hbm.at[pl.ds(c * NPAD + s * RPT, RPT)],
        )

    return deg_kernel(col)


def _sc_scatter(row, col, h2):
    @functools.partial(
        pl.kernel,
        out_type=jax.ShapeDtypeStruct((2 * NPAD, D), jnp.float32),
        mesh=_vmesh(),
        scratch_types=[
            pltpu.VMEM_SHARED((NPAD, D), jnp.float32),
            pltpu.VMEM((128, D), jnp.float32),
            pltpu.VMEM((CHUNK,), jnp.int32),
            pltpu.VMEM((CHUNK,), jnp.int32),
            pltpu.VMEM((CHUNK, D), jnp.float32),
            pltpu.SemaphoreType.DMA,
        ],
    )
    def scat_kernel(row_hbm, col_hbm, h2_hbm, out_hbm, acc, zbuf, rowi, coli, rows, sem):
        c = lax.axis_index("c")
        s = lax.axis_index("s")
        wid = s * 2 + c

        @pl.loop(0, 128)
        def _(i):
            @pl.loop(0, D, step=16)
            def _(j):
                zbuf[i, pl.ds(j, 16)] = jnp.zeros((16,), jnp.float32)

        @pl.loop(0, RPT // 128)
        def _(k):
            pltpu.sync_copy(zbuf, acc.at[pl.ds(s * RPT + k * 128, 128)])

        plsc.subcore_barrier()

        @pl.loop(0, CHUNK_ITERS)
        def _(i):
            j = wid + i * TILES

            @pl.when(j < NUM_CHUNKS)
            def _():
                pltpu.sync_copy(row_hbm.at[pl.ds(j * CHUNK, CHUNK)], rowi)
                pltpu.sync_copy(col_hbm.at[pl.ds(j * CHUNK, CHUNK)], coli)
                pltpu.async_copy(h2_hbm.at[rowi], rows, sem).wait()
                pltpu.sync_copy(rows, acc.at[coli], add=True)

        plsc.subcore_barrier()
        pltpu.sync_copy(
            acc.at[pl.ds(s * RPT, RPT)],
            out_hbm.at[pl.ds(c * NPAD + s * RPT, RPT)],
        )

    return scat_kernel(row, col, h2)


def _tc_h2(x, w, degp):
    def body(x_ref, w_ref, d0_ref, d1_ref, h2_ref):
        d = d0_ref[:, 0:1] + d1_ref[:, 0:1] + 1.0
        dis = lax.rsqrt(d)
        h = jnp.dot(x_ref[...], w_ref[...], preferred_element_type=jnp.float32)
        h2_ref[...] = h * dis

    return pl.pallas_call(
        body,
        grid=(GRID,),
        in_specs=[
            pl.BlockSpec((BLK, 128), lambda i: (i, 0)),
            pl.BlockSpec((128, 128), lambda i: (0, 0)),
            pl.BlockSpec((BLK, 16), lambda i: (i, 0)),
            pl.BlockSpec((BLK, 16), lambda i: (i + GRID, 0)),
        ],
        out_specs=pl.BlockSpec((BLK, 128), lambda i: (i, 0)),
        out_shape=jax.ShapeDtypeStruct((N, 128), jnp.float32),
    )(x, w, degp, degp)


def _tc_final(tmp, h2, degp, bmu, bls):
    def body(t0, t1, h2r, d0, d1, bm, bl, mu_ref, ls_ref):
        d = d0[:, 0:1] + d1[:, 0:1] + 1.0
        dis = lax.rsqrt(d)
        out = dis * (t0[...] + t1[...] + h2r[...])
        mu_ref[...] = out[:, :64] + bm[0:1, :]
        ls_ref[...] = out[:, 64:] + bl[0:1, :]

    return pl.pallas_call(
        body,
        grid=(GRID,),
        in_specs=[
            pl.BlockSpec((BLK, 128), lambda i: (i, 0)),
            pl.BlockSpec((BLK, 128), lambda i: (i + GRID, 0)),
            pl.BlockSpec((BLK, 128), lambda i: (i, 0)),
            pl.BlockSpec((BLK, 16), lambda i: (i, 0)),
            pl.BlockSpec((BLK, 16), lambda i: (i + GRID, 0)),
            pl.BlockSpec((8, 64), lambda i: (0, 0)),
            pl.BlockSpec((8, 64), lambda i: (0, 0)),
        ],
        out_specs=[
            pl.BlockSpec((BLK, 64), lambda i: (i, 0)),
            pl.BlockSpec((BLK, 64), lambda i: (i, 0)),
        ],
        out_shape=[
            jax.ShapeDtypeStruct((N, 64), jnp.float32),
            jax.ShapeDtypeStruct((N, 64), jnp.float32),
        ],
    )(tmp, tmp, h2, degp, degp, bmu, bls)


def kernel(x, edge_index, W_mu, b_mu, W_logstd, b_logstd):
    w = jnp.concatenate([W_mu, W_logstd], axis=1)
    row = edge_index[0]
    col = edge_index[1]
    bmu = jnp.tile(b_mu[None, :], (8, 1))
    bls = jnp.tile(b_logstd[None, :], (8, 1))
    degp = _sc_degree(col)
    h2 = _tc_h2(x, w, degp)
    tmp = _sc_scatter(row, col, h2)
    mu, logstd = _tc_final(tmp, h2, degp, bmu, bls)
    return (mu, logstd)
